# baseline (speedup 1.0000x reference)
import jax
import jax.numpy as jnp
from jax.experimental import pallas as pl
from jax.experimental.pallas import tpu as pltpu

_N = 6400000
_GRID = 4
_ROWS = _N // (_GRID * 128)
_LANES = 128


def _copy_body(in_ref, out_ref):
    out_ref[...] = in_ref[...]


def kernel(Zj):
    x = Zj.reshape(_GRID, _ROWS, _LANES)
    y = pl.pallas_call(
        _copy_body,
        out_shape=jax.ShapeDtypeStruct((_GRID, _ROWS, _LANES), Zj.dtype),
        grid=(_GRID,),
        in_specs=[pl.BlockSpec((1, _ROWS, _LANES), lambda i: (i, 0, 0))],
        out_specs=pl.BlockSpec((1, _ROWS, _LANES), lambda i: (i, 0, 0)),
    )(x)
    return y.reshape(_N)


# speedup vs baseline: 3.3535x; 3.3535x over previous
import jax
import jax.numpy as jnp
from jax.experimental import pallas as pl
from jax.experimental.pallas import tpu as pltpu

_N = 6400000
_LANES = 128
_ROWS = _N // _LANES
_NCHUNK = 25
_CROWS = _ROWS // _NCHUNK
_NBUF = 8
_LAG = 4


def _copy_body(in_ref, out_ref, bufs, in_sems, out_sems):
    def in_copy(i, b):
        return pltpu.make_async_copy(
            in_ref.at[pl.ds(i * _CROWS, _CROWS)], bufs.at[b], in_sems.at[b])

    def out_copy(i, b):
        return pltpu.make_async_copy(
            bufs.at[b], out_ref.at[pl.ds(i * _CROWS, _CROWS)], out_sems.at[b])

    for i in range(_NCHUNK + _LAG):
        if i < _NCHUNK:
            b = i % _NBUF
            if i >= _NBUF:
                out_copy(i - _NBUF, b).wait()
            in_copy(i, b).start()
        j = i - _LAG
        if j >= 0:
            bj = j % _NBUF
            in_copy(j, bj).wait()
            out_copy(j, bj).start()
    for j in range(_NCHUNK - _NBUF, _NCHUNK):
        out_copy(j, j % _NBUF).wait()


def kernel(Zj):
    x = Zj.reshape(_ROWS, _LANES)
    y = pl.pallas_call(
        _copy_body,
        out_shape=jax.ShapeDtypeStruct((_ROWS, _LANES), Zj.dtype),
        in_specs=[pl.BlockSpec(memory_space=pl.ANY)],
        out_specs=pl.BlockSpec(memory_space=pl.ANY),
        scratch_shapes=[
            pltpu.VMEM((_NBUF, _CROWS, _LANES), jnp.float32),
            pltpu.SemaphoreType.DMA((_NBUF,)),
            pltpu.SemaphoreType.DMA((_NBUF,)),
        ],
    )(x)
    return y.reshape(_N)


# speedup vs baseline: 3.3566x; 1.0009x over previous
import jax
import jax.numpy as jnp
from jax.experimental import pallas as pl
from jax.experimental.pallas import tpu as pltpu

_N = 6400000
_LANES = 128
_ROWS = _N // _LANES
_NCHUNK = 10
_CROWS = _ROWS // _NCHUNK
_NBUF = 4
_LAG = 2


def _copy_body(in_ref, out_ref, bufs, in_sems, out_sems):
    def in_copy(i, b):
        return pltpu.make_async_copy(
            in_ref.at[pl.ds(i * _CROWS, _CROWS)], bufs.at[b], in_sems.at[b])

    def out_copy(i, b):
        return pltpu.make_async_copy(
            bufs.at[b], out_ref.at[pl.ds(i * _CROWS, _CROWS)], out_sems.at[b])

    for i in range(_NCHUNK + _LAG):
        if i < _NCHUNK:
            b = i % _NBUF
            if i >= _NBUF:
                out_copy(i - _NBUF, b).wait()
            in_copy(i, b).start()
        j = i - _LAG
        if j >= 0:
            bj = j % _NBUF
            in_copy(j, bj).wait()
            out_copy(j, bj).start()
    for j in range(_NCHUNK - _NBUF, _NCHUNK):
        out_copy(j, j % _NBUF).wait()


def kernel(Zj):
    x = Zj.reshape(_ROWS, _LANES)
    y = pl.pallas_call(
        _copy_body,
        out_shape=jax.ShapeDtypeStruct((_ROWS, _LANES), Zj.dtype),
        in_specs=[pl.BlockSpec(memory_space=pl.ANY)],
        out_specs=pl.BlockSpec(memory_space=pl.ANY),
        scratch_shapes=[
            pltpu.VMEM((_NBUF, _CROWS, _LANES), jnp.float32),
            pltpu.SemaphoreType.DMA((_NBUF,)),
            pltpu.SemaphoreType.DMA((_NBUF,)),
        ],
    )(x)
    return y.reshape(_N)


# speedup vs baseline: 3.3577x; 1.0003x over previous
import jax
import jax.numpy as jnp
from jax.experimental import pallas as pl
from jax.experimental.pallas import tpu as pltpu

_N = 6400000
_LANES = 128
_ROWS = _N // _LANES
_NCHUNK = 5
_CROWS = _ROWS // _NCHUNK
_NBUF = 3
_LAG = 1


def _copy_body(in_ref, out_ref, bufs, in_sems, out_sems):
    def in_copy(i, b):
        return pltpu.make_async_copy(
            in_ref.at[pl.ds(i * _CROWS, _CROWS)], bufs.at[b], in_sems.at[b])

    def out_copy(i, b):
        return pltpu.make_async_copy(
            bufs.at[b], out_ref.at[pl.ds(i * _CROWS, _CROWS)], out_sems.at[b])

    for i in range(_NCHUNK + _LAG):
        if i < _NCHUNK:
            b = i % _NBUF
            if i >= _NBUF:
                out_copy(i - _NBUF, b).wait()
            in_copy(i, b).start()
        j = i - _LAG
        if j >= 0:
            bj = j % _NBUF
            in_copy(j, bj).wait()
            out_copy(j, bj).start()
    for j in range(_NCHUNK - _NBUF, _NCHUNK):
        out_copy(j, j % _NBUF).wait()


def kernel(Zj):
    x = Zj.reshape(_ROWS, _LANES)
    y = pl.pallas_call(
        _copy_body,
        out_shape=jax.ShapeDtypeStruct((_ROWS, _LANES), Zj.dtype),
        in_specs=[pl.BlockSpec(memory_space=pl.ANY)],
        out_specs=pl.BlockSpec(memory_space=pl.ANY),
        scratch_shapes=[
            pltpu.VMEM((_NBUF, _CROWS, _LANES), jnp.float32),
            pltpu.SemaphoreType.DMA((_NBUF,)),
            pltpu.SemaphoreType.DMA((_NBUF,)),
        ],
    )(x)
    return y.reshape(_N)
